# baseline (speedup 1.0000x reference)
import jax
import jax.numpy as jnp
import numpy as np
from jax.experimental import pallas as pl
from jax.experimental.pallas import tpu as pltpu

_B, _N, _P, _K = 8, 8192, 64, 256
_C = 128
_NC = _N // _C


def _lanesum(acc):
    return jnp.sum(acc, axis=1, keepdims=True)


def _count_le(bits_ref, mid):
    accs = []
    for j in range(_NC):
        c = (bits_ref[:, j * _C:(j + 1) * _C] <= mid).astype(jnp.int32)
        if j < 4:
            accs.append(c)
        else:
            accs[j % 4] = accs[j % 4] + c
    return _lanesum((accs[0] + accs[1]) + (accs[2] + accs[3]))


def _kth_thresh(bits_ref, seed=None):
    if seed is None:
        lo0 = jnp.zeros((_P, 1), jnp.int32)
        hi0 = jnp.full((_P, 1), 0x7F7FFFFF, jnp.int32)
    else:
        lo_s = jnp.maximum(seed - (2 << 23), 0)
        hi_s = jnp.minimum(seed + (2 << 23), 0x7F7FFFFF)
        lo0 = jnp.where(_count_le(bits_ref, lo_s) < _K, lo_s, 0)
        hi0 = jnp.where(_count_le(bits_ref, hi_s) >= _K, hi_s,
                        jnp.int32(0x7F7FFFFF))
    t0 = hi0
    found0 = jnp.zeros((_P, 1), jnp.int32)

    def cond(carry):
        lo, hi, t, found = carry
        busy = jnp.where((found == 0) & (hi - lo > 1), 1, 0)
        return jnp.max(busy) > 0

    def body(carry):
        lo, hi, t, found = carry
        upd = (found == 0) & (hi - lo > 1)
        mid = lo + jax.lax.shift_right_logical(hi - lo, 1)
        cnt = _count_le(bits_ref, mid)
        ge = cnt >= _K
        eq = cnt == _K
        hi2 = jnp.where(upd & ge, mid, hi)
        lo2 = jnp.where(upd & jnp.logical_not(ge), mid, lo)
        t2 = jnp.where(upd & eq, mid, t)
        found2 = jnp.where(upd & eq, 1, found)
        return lo2, hi2, t2, found2

    lo, hi, t, found = jax.lax.while_loop(cond, body, (lo0, hi0, t0, found0))
    return jnp.where(found != 0, t, hi)


def _d2_bits_store(out_ref, x0, x1, x2, c0, c1, c2):
    for j in range(_NC):
        sl = slice(j * _C, (j + 1) * _C)
        d2 = ((x0[:, sl] - c0) ** 2 + (x1[:, sl] - c1) ** 2
              + (x2[:, sl] - c2) ** 2)
        out_ref[:, sl] = jax.lax.bitcast_convert_type(d2, jnp.int32)


def _mask_means(bits_ref, t, x0, x1, x2):
    a_c = a_0 = a_1 = a_2 = jnp.zeros((_P, _C), jnp.float32)
    for j in range(_NC):
        sl = slice(j * _C, (j + 1) * _C)
        m = (bits_ref[:, sl] <= t).astype(jnp.float32)
        a_c = a_c + m
        a_0 = a_0 + m * x0[:, sl]
        a_1 = a_1 + m * x1[:, sl]
        a_2 = a_2 + m * x2[:, sl]
    cnt = _lanesum(a_c)
    inv = 1.0 / cnt
    return cnt, _lanesum(a_0) * inv, _lanesum(a_1) * inv, _lanesum(a_2) * inv


def _mask_cov(bits_ref, t, x0, x1, x2, g0, g1, g2):
    z = jnp.zeros((_P, _C), jnp.float32)
    a_c = s0 = s1 = s2 = z
    q00 = q01 = q02 = z
    for j in range(_NC):
        sl = slice(j * _C, (j + 1) * _C)
        m = (bits_ref[:, sl] <= t).astype(jnp.float32)
        y0 = x0[:, sl] - g0
        w0 = m * y0
        a_c = a_c + m
        s0 = s0 + w0
        q00 = q00 + w0 * y0
        q01 = q01 + w0 * (x1[:, sl] - g1)
        q02 = q02 + w0 * (x2[:, sl] - g2)
    q11 = q12 = q22 = z
    for j in range(_NC):
        sl = slice(j * _C, (j + 1) * _C)
        m = (bits_ref[:, sl] <= t).astype(jnp.float32)
        y1 = x1[:, sl] - g1
        w1 = m * y1
        s1 = s1 + w1
        q11 = q11 + w1 * y1
        q12 = q12 + w1 * (x2[:, sl] - g2)
        y2 = x2[:, sl] - g2
        w2 = m * y2
        s2 = s2 + w2
        q22 = q22 + w2 * y2
    icnt = 1.0 / _lanesum(a_c)
    d0 = _lanesum(s0) * icnt
    d1 = _lanesum(s1) * icnt
    d2m = _lanesum(s2) * icnt
    return (_lanesum(q00) * icnt - d0 * d0,
            _lanesum(q01) * icnt - d0 * d1,
            _lanesum(q02) * icnt - d0 * d2m,
            _lanesum(q11) * icnt - d1 * d1,
            _lanesum(q12) * icnt - d1 * d2m,
            _lanesum(q22) * icnt - d2m * d2m)


def _body(pct_ref, cen_ref, elong_ref, smooth_ref, bits_ref, bits2_ref):
    x0 = pct_ref[0, 0:1, :]
    x1 = pct_ref[0, 1:2, :]
    x2 = pct_ref[0, 2:3, :]
    c0 = cen_ref[0, :, 0:1]
    c1 = cen_ref[0, :, 1:2]
    c2 = cen_ref[0, :, 2:3]

    _d2_bits_store(bits_ref, x0, x1, x2, c0, c1, c2)
    t1 = _kth_thresh(bits_ref)
    _, g0, g1, g2 = _mask_means(bits_ref, t1, x0, x1, x2)

    _d2_bits_store(bits2_ref, x0, x1, x2, g0, g1, g2)
    t2 = _kth_thresh(bits2_ref, seed=t1)
    a00, a01, a02, a11, a12, a22 = _mask_cov(
        bits2_ref, t2, x0, x1, x2, g0, g1, g2)

    q = (a00 + a11 + a22) / 3.0
    p1 = a01 * a01 + a02 * a02 + a12 * a12
    b00 = a00 - q
    b11 = a11 - q
    b22 = a22 - q
    p2 = b00 * b00 + b11 * b11 + b22 * b22 + 2.0 * p1
    p = jnp.sqrt(p2 / 6.0) + 1e-30
    ip = 1.0 / p
    c00b = b00 * ip
    c11b = b11 * ip
    c22b = b22 * ip
    c01b = a01 * ip
    c02b = a02 * ip
    c12b = a12 * ip
    det = (c00b * (c11b * c22b - c12b * c12b)
           - c01b * (c01b * c22b - c12b * c02b)
           + c02b * (c01b * c12b - c11b * c02b))
    r = jnp.clip(0.5 * det, -1.0, 1.0)
    c = 0.5 + 0.5 * jnp.sqrt(jnp.maximum(0.5 * (r + 1.0), 0.0))
    for _ in range(10):
        g = (4.0 * c * c - 3.0) * c - r
        dg = 12.0 * c * c - 3.0
        c = jnp.clip(c - g / (dg + 1e-12), 0.5, 1.0)
    s = jnp.sqrt(jnp.maximum(1.0 - c * c, 0.0))
    lmax = q + 2.0 * p * c
    lmin = q + 2.0 * p * (-0.5 * c - 0.8660254037844386 * s)
    lmid = 3.0 * q - lmax - lmin
    trace = a00 + a11 + a22
    elong = (lmax - lmid) / (trace + 1e-9)
    elong_ref[0] = jnp.sum(elong, axis=0, keepdims=True)

    e00 = a00 - lmin
    e11 = a11 - lmin
    e22 = a22 - lmin
    f00 = a00 - lmid
    f11 = a11 - lmid
    f22 = a22 - lmid
    m00 = e00 * f00 + a01 * a01 + a02 * a02
    m10 = a01 * f00 + e11 * a01 + a12 * a02
    m20 = a02 * f00 + a12 * a01 + e22 * a02
    m01 = e00 * a01 + a01 * f11 + a02 * a12
    m11 = a01 * a01 + e11 * f11 + a12 * a12
    m21 = a02 * a01 + a12 * f11 + e22 * a12
    m02 = e00 * a02 + a01 * a12 + a02 * f22
    m12 = a01 * a02 + e11 * a12 + a12 * f22
    m22 = a02 * a02 + a12 * a12 + e22 * f22
    nc0 = m00 * m00 + m10 * m10 + m20 * m20
    nc1 = m01 * m01 + m11 * m11 + m21 * m21
    nc2 = m02 * m02 + m12 * m12 + m22 * m22
    use1 = nc1 > nc0
    vx = jnp.where(use1, m01, m00)
    vy = jnp.where(use1, m11, m10)
    vz = jnp.where(use1, m21, m20)
    use2 = nc2 > jnp.maximum(nc0, nc1)
    vx = jnp.where(use2, m02, vx)
    vy = jnp.where(use2, m12, vy)
    vz = jnp.where(use2, m22, vz)
    inv = 1.0 / (jnp.sqrt(vx * vx + vy * vy + vz * vz) + 1e-9)
    ux = vx * inv
    uy = vy * inv
    uz = vz * inv
    cosv = (ux[:-1] * ux[1:] + uy[:-1] * uy[1:] + uz[:-1] * uz[1:])
    smooth_ref[0] = jnp.sum(1.0 - cosv * cosv, axis=0, keepdims=True)


def kernel(pointclouds):
    pct = jnp.transpose(pointclouds, (0, 2, 1))
    cidx = jnp.clip(
        jnp.round(jnp.linspace(0, _N - 1, _P)).astype(jnp.int32), 0, _N - 1)
    centers = pointclouds[:, cidx, :]
    elong, smooth = pl.pallas_call(
        _body,
        grid=(_B,),
        in_specs=[
            pl.BlockSpec((1, 3, _N), lambda b: (b, 0, 0)),
            pl.BlockSpec((1, _P, 3), lambda b: (b, 0, 0)),
        ],
        out_specs=[
            pl.BlockSpec((1, 1, 1), lambda b: (b, 0, 0)),
            pl.BlockSpec((1, 1, 1), lambda b: (b, 0, 0)),
        ],
        out_shape=[
            jax.ShapeDtypeStruct((_B, 1, 1), jnp.float32),
            jax.ShapeDtypeStruct((_B, 1, 1), jnp.float32),
        ],
        scratch_shapes=[
            pltpu.VMEM((_P, _N), jnp.int32),
            pltpu.VMEM((_P, _N), jnp.int32),
        ],
    )(pct, centers)
    loss = -jnp.sum(elong) / _B + jnp.sum(smooth) / (_B * (_P - 1))
    return loss


# speedup vs baseline: 1.0075x; 1.0075x over previous
import jax
import jax.numpy as jnp
import numpy as np
from jax.experimental import pallas as pl
from jax.experimental.pallas import tpu as pltpu

_B, _N, _P, _K = 8, 8192, 64, 256
_C = 128
_NC = _N // _C


def _lanesum(acc):
    return jnp.sum(acc, axis=1, keepdims=True)


def _count_le(bits_ref, mid):
    accs = []
    for j in range(_NC):
        c = (bits_ref[:, j * _C:(j + 1) * _C] <= mid).astype(jnp.int32)
        if j < 4:
            accs.append(c)
        else:
            accs[j % 4] = accs[j % 4] + c
    return _lanesum((accs[0] + accs[1]) + (accs[2] + accs[3]))


def _kth_thresh(bits_ref, seed=None):
    if seed is None:
        lo0 = jnp.zeros((_P, 1), jnp.int32)
        hi0 = jnp.full((_P, 1), 0x7F7FFFFF, jnp.int32)
    else:
        lo_s = jnp.maximum(seed - (2 << 23), 0)
        hi_s = jnp.minimum(seed + (2 << 23), 0x7F7FFFFF)
        lo0 = jnp.where(_count_le(bits_ref, lo_s) < _K, lo_s, 0)
        hi0 = jnp.where(_count_le(bits_ref, hi_s) >= _K, hi_s,
                        jnp.int32(0x7F7FFFFF))
    t0 = hi0
    found0 = jnp.zeros((_P, 1), jnp.int32)

    def cond(carry):
        lo, hi, t, found = carry
        busy = jnp.where((found == 0) & (hi - lo > 1), 1, 0)
        return jnp.max(busy) > 0

    def body(carry):
        lo, hi, t, found = carry
        upd = (found == 0) & (hi - lo > 1)
        mid = lo + jax.lax.shift_right_logical(hi - lo, 1)
        cnt = _count_le(bits_ref, mid)
        ge = cnt >= _K
        eq = cnt == _K
        hi2 = jnp.where(upd & ge, mid, hi)
        lo2 = jnp.where(upd & jnp.logical_not(ge), mid, lo)
        t2 = jnp.where(upd & eq, mid, t)
        found2 = jnp.where(upd & eq, 1, found)
        return lo2, hi2, t2, found2

    lo, hi, t, found = jax.lax.while_loop(cond, body, (lo0, hi0, t0, found0))
    return jnp.where(found != 0, t, hi)


def _d2_bits_store(out_ref, x0, x1, x2, c0, c1, c2):
    for j in range(_NC):
        sl = slice(j * _C, (j + 1) * _C)
        d2 = ((x0[:, sl] - c0) ** 2 + (x1[:, sl] - c1) ** 2
              + (x2[:, sl] - c2) ** 2)
        out_ref[:, sl] = jax.lax.bitcast_convert_type(d2, jnp.int32)


def _mask_means(bits_ref, t, x0, x1, x2):
    a_c = a_0 = a_1 = a_2 = jnp.zeros((_P, _C), jnp.float32)
    for j in range(_NC):
        sl = slice(j * _C, (j + 1) * _C)
        m = (bits_ref[:, sl] <= t).astype(jnp.float32)
        a_c = a_c + m
        a_0 = a_0 + m * x0[:, sl]
        a_1 = a_1 + m * x1[:, sl]
        a_2 = a_2 + m * x2[:, sl]
    cnt = _lanesum(a_c)
    inv = 1.0 / cnt
    return cnt, _lanesum(a_0) * inv, _lanesum(a_1) * inv, _lanesum(a_2) * inv


def _mask_cov(bits_ref, t, x0, x1, x2, n0, n1, n2, icnt):
    z = jnp.zeros((_P, _C), jnp.float32)
    a00 = a01 = a02 = z
    for j in range(_NC):
        sl = slice(j * _C, (j + 1) * _C)
        m = (bits_ref[:, sl] <= t).astype(jnp.float32)
        w0 = m * (x0[:, sl] - n0)
        a00 = a00 + w0 * (x0[:, sl] - n0)
        a01 = a01 + w0 * (x1[:, sl] - n1)
        a02 = a02 + w0 * (x2[:, sl] - n2)
    a11 = a12 = a22 = z
    for j in range(_NC):
        sl = slice(j * _C, (j + 1) * _C)
        m = (bits_ref[:, sl] <= t).astype(jnp.float32)
        y1 = x1[:, sl] - n1
        w1 = m * y1
        a11 = a11 + w1 * y1
        a12 = a12 + w1 * (x2[:, sl] - n2)
        y2 = x2[:, sl] - n2
        a22 = a22 + (m * y2) * y2
    return (_lanesum(a00) * icnt, _lanesum(a01) * icnt, _lanesum(a02) * icnt,
            _lanesum(a11) * icnt, _lanesum(a12) * icnt, _lanesum(a22) * icnt)


def _body(pct_ref, cen_ref, elong_ref, smooth_ref, bits_ref, bits2_ref):
    x0 = pct_ref[0, 0:1, :]
    x1 = pct_ref[0, 1:2, :]
    x2 = pct_ref[0, 2:3, :]
    c0 = cen_ref[0, :, 0:1]
    c1 = cen_ref[0, :, 1:2]
    c2 = cen_ref[0, :, 2:3]

    _d2_bits_store(bits_ref, x0, x1, x2, c0, c1, c2)
    t1 = _kth_thresh(bits_ref)
    _, g0, g1, g2 = _mask_means(bits_ref, t1, x0, x1, x2)

    _d2_bits_store(bits2_ref, x0, x1, x2, g0, g1, g2)
    t2 = _kth_thresh(bits2_ref, seed=t1)
    cnt2, n0, n1, n2 = _mask_means(bits2_ref, t2, x0, x1, x2)
    icnt = 1.0 / cnt2
    a00, a01, a02, a11, a12, a22 = _mask_cov(
        bits2_ref, t2, x0, x1, x2, n0, n1, n2, icnt)

    q = (a00 + a11 + a22) / 3.0
    p1 = a01 * a01 + a02 * a02 + a12 * a12
    b00 = a00 - q
    b11 = a11 - q
    b22 = a22 - q
    p2 = b00 * b00 + b11 * b11 + b22 * b22 + 2.0 * p1
    p = jnp.sqrt(p2 / 6.0) + 1e-30
    ip = 1.0 / p
    c00b = b00 * ip
    c11b = b11 * ip
    c22b = b22 * ip
    c01b = a01 * ip
    c02b = a02 * ip
    c12b = a12 * ip
    det = (c00b * (c11b * c22b - c12b * c12b)
           - c01b * (c01b * c22b - c12b * c02b)
           + c02b * (c01b * c12b - c11b * c02b))
    r = jnp.clip(0.5 * det, -1.0, 1.0)
    c = 0.5 + 0.5 * jnp.sqrt(jnp.maximum(0.5 * (r + 1.0), 0.0))
    for _ in range(10):
        g = (4.0 * c * c - 3.0) * c - r
        dg = 12.0 * c * c - 3.0
        c = jnp.clip(c - g / (dg + 1e-12), 0.5, 1.0)
    s = jnp.sqrt(jnp.maximum(1.0 - c * c, 0.0))
    lmax = q + 2.0 * p * c
    lmin = q + 2.0 * p * (-0.5 * c - 0.8660254037844386 * s)
    lmid = 3.0 * q - lmax - lmin
    trace = a00 + a11 + a22
    elong = (lmax - lmid) / (trace + 1e-9)
    elong_ref[0] = jnp.sum(elong, axis=0, keepdims=True)

    e00 = a00 - lmin
    e11 = a11 - lmin
    e22 = a22 - lmin
    f00 = a00 - lmid
    f11 = a11 - lmid
    f22 = a22 - lmid
    m00 = e00 * f00 + a01 * a01 + a02 * a02
    m10 = a01 * f00 + e11 * a01 + a12 * a02
    m20 = a02 * f00 + a12 * a01 + e22 * a02
    m01 = e00 * a01 + a01 * f11 + a02 * a12
    m11 = a01 * a01 + e11 * f11 + a12 * a12
    m21 = a02 * a01 + a12 * f11 + e22 * a12
    m02 = e00 * a02 + a01 * a12 + a02 * f22
    m12 = a01 * a02 + e11 * a12 + a12 * f22
    m22 = a02 * a02 + a12 * a12 + e22 * f22
    nc0 = m00 * m00 + m10 * m10 + m20 * m20
    nc1 = m01 * m01 + m11 * m11 + m21 * m21
    nc2 = m02 * m02 + m12 * m12 + m22 * m22
    use1 = nc1 > nc0
    vx = jnp.where(use1, m01, m00)
    vy = jnp.where(use1, m11, m10)
    vz = jnp.where(use1, m21, m20)
    use2 = nc2 > jnp.maximum(nc0, nc1)
    vx = jnp.where(use2, m02, vx)
    vy = jnp.where(use2, m12, vy)
    vz = jnp.where(use2, m22, vz)
    inv = 1.0 / (jnp.sqrt(vx * vx + vy * vy + vz * vz) + 1e-9)
    ux = vx * inv
    uy = vy * inv
    uz = vz * inv
    cosv = (ux[:-1] * ux[1:] + uy[:-1] * uy[1:] + uz[:-1] * uz[1:])
    smooth_ref[0] = jnp.sum(1.0 - cosv * cosv, axis=0, keepdims=True)


def kernel(pointclouds):
    pct = jnp.transpose(pointclouds, (0, 2, 1))
    cidx = jnp.clip(
        jnp.round(jnp.linspace(0, _N - 1, _P)).astype(jnp.int32), 0, _N - 1)
    centers = pointclouds[:, cidx, :]
    elong, smooth = pl.pallas_call(
        _body,
        grid=(_B,),
        in_specs=[
            pl.BlockSpec((1, 3, _N), lambda b: (b, 0, 0)),
            pl.BlockSpec((1, _P, 3), lambda b: (b, 0, 0)),
        ],
        out_specs=[
            pl.BlockSpec((1, 1, 1), lambda b: (b, 0, 0)),
            pl.BlockSpec((1, 1, 1), lambda b: (b, 0, 0)),
        ],
        out_shape=[
            jax.ShapeDtypeStruct((_B, 1, 1), jnp.float32),
            jax.ShapeDtypeStruct((_B, 1, 1), jnp.float32),
        ],
        scratch_shapes=[
            pltpu.VMEM((_P, _N), jnp.int32),
            pltpu.VMEM((_P, _N), jnp.int32),
        ],
    )(pct, centers)
    loss = -jnp.sum(elong) / _B + jnp.sum(smooth) / (_B * (_P - 1))
    return loss
